# Initial kernel scaffold; baseline (speedup 1.0000x reference)
import jax
import jax.numpy as jnp
from jax.experimental import pallas as pl


def kernel(v_feats, g_feats, spt_feats, uni_ids, ids):
    raise NotImplementedError("write your pallas kernel here")


# speedup vs baseline: 18.7541x; 18.7541x over previous
import functools

import jax
import jax.numpy as jnp
from jax.experimental import pallas as pl
from jax.experimental.pallas import tpu as pltpu

_INV_EPS = 20.0
_SINK_EPS = 1e-9


def _norm_body(v_ref, o_ref):
    x = v_ref[...]
    n = jnp.sqrt(jnp.sum(x * x, axis=1, keepdims=True))
    o_ref[...] = x / jnp.maximum(n, 1e-12)


def _gram_body(ti_r, tj_r, num_r, vi_r, vj_r, rs_r, re_r, k0_r, a1_r, y_r, *, B, T):
    t = pl.program_id(0)

    @pl.when(t == 0)
    def _zero():
        y_r[...] = jnp.zeros_like(y_r)

    @pl.when(t < num_r[0])
    def _compute():
        s = jax.lax.dot_general(
            vi_r[...], vj_r[...], (((1,), (1,)), ((), ())),
            preferred_element_type=jnp.float32)
        cols = jax.lax.broadcasted_iota(jnp.int32, (B, B), 1) + tj_r[t] * B
        rs_b = rs_r[...].reshape(B, 1)
        re_b = re_r[...].reshape(B, 1)
        m = (cols >= rs_b) & (cols < re_b)
        k0 = jnp.where(m, jnp.exp((s - 1.0) * _INV_EPS), 0.0)
        k0_r[0] = k0
        bi = ti_r[t]
        cur = y_r[pl.ds(bi, 1)]
        y_r[pl.ds(bi, 1)] = cur + jnp.sum(k0, axis=1, keepdims=True)[None]

    @pl.when(t == T - 1)
    def _emit():
        a1_r[...] = 1.0 / (y_r[...] + _SINK_EPS)


def _matvec_body(ti_r, tj_r, num_r, k0_r, x_r, s_r, snew_r, y_r, *, T):
    t = pl.program_id(0)

    @pl.when(t == 0)
    def _zero():
        y_r[...] = jnp.zeros_like(y_r)

    @pl.when(t < num_r[0])
    def _compute():
        bi = ti_r[t]
        xb = x_r[pl.ds(tj_r[t], 1)].reshape(x_r.shape[1], 1)
        yt = jax.lax.dot_general(
            k0_r[0], xb, (((1,), (0,)), ((), ())),
            preferred_element_type=jnp.float32)
        cur = y_r[pl.ds(bi, 1)]
        y_r[pl.ds(bi, 1)] = cur + yt[None]

    @pl.when(t == T - 1)
    def _emit():
        snew_r[...] = s_r[...] / (s_r[...] * y_r[...] + _SINK_EPS)


def _out_body(ti_r, tj_r, num_r, k0_r, b_r, v_r, o_r, pf_r, *, B, T):
    t = pl.program_id(0)
    bi = ti_r[t]
    prv = ti_r[jnp.maximum(t - 1, 0)]
    nxt = ti_r[jnp.minimum(t + 1, T - 1)]
    isfirst = (t == 0) | (prv != bi)
    islast = (t >= num_r[0] - 1) | (nxt != bi)

    @pl.when(t < num_r[0])
    def _compute():
        k0 = k0_r[0]
        xb = b_r[pl.ds(tj_r[t], 1)].reshape(B, 1)
        pft = jax.lax.dot_general(
            k0, xb, (((1,), (0,)), ((), ())),
            preferred_element_type=jnp.float32)
        vs = (v_r[...] * xb).astype(jnp.bfloat16)
        contrib = jax.lax.dot_general(
            k0.astype(jnp.bfloat16), vs, (((1,), (0,)), ((), ())),
            preferred_element_type=jnp.float32)

        @pl.when(isfirst)
        def _init():
            pf_r[...] = pft
            o_r[...] = contrib

        @pl.when(jnp.logical_not(isfirst))
        def _acc():
            pf_r[...] = pf_r[...] + pft
            o_r[...] = o_r[...] + contrib

        @pl.when(islast)
        def _final():
            o_r[...] = o_r[...] * (1.0 / pf_r[...])


def kernel(v_feats, g_feats, spt_feats, uni_ids, ids):
    N, d = v_feats.shape
    G = uni_ids.shape[0]
    B = 256 if N % 256 == 0 and N >= 2048 else 64
    NB = N // B
    T = NB * NB

    ids32 = ids.astype(jnp.int32)
    gstart = jnp.searchsorted(ids32, jnp.arange(G, dtype=jnp.int32),
                              side="left").astype(jnp.int32)
    gend = jnp.searchsorted(ids32, jnp.arange(G, dtype=jnp.int32),
                            side="right").astype(jnp.int32)
    rs = gstart[ids32].reshape(NB, B, 1)
    re = gend[ids32].reshape(NB, B, 1)

    idsb = ids32.reshape(NB, B)
    first = idsb[:, 0]
    last = idsb[:, -1]
    act = (first[:, None] <= last[None, :]) & (first[None, :] <= last[:, None])
    flat = act.reshape(-1)
    order = jnp.argsort(jnp.logical_not(flat), stable=True).astype(jnp.int32)
    num = flat.sum().astype(jnp.int32)
    sel = order[jnp.minimum(jnp.arange(T, dtype=jnp.int32), num - 1)]
    ti = (sel // NB).astype(jnp.int32)
    tj = (sel % NB).astype(jnp.int32)
    num_arr = num.reshape(1)

    f32 = jnp.float32

    v_n = pl.pallas_call(
        _norm_body,
        grid=(NB,),
        in_specs=[pl.BlockSpec((B, d), lambda i: (i, 0))],
        out_specs=pl.BlockSpec((B, d), lambda i: (i, 0)),
        out_shape=jax.ShapeDtypeStruct((N, d), f32),
    )(v_feats)

    def row_map(t, ti, tj, nm):
        return (ti[t], 0)

    def col_map(t, ti, tj, nm):
        return (tj[t], 0)

    def row3_map(t, ti, tj, nm):
        return (ti[t], 0, 0)

    def tile_map(t, ti, tj, nm):
        return (jnp.minimum(t, nm[0] - 1), 0, 0)

    def whole3_map(t, ti, tj, nm):
        return (0, 0, 0)

    k0, a1 = pl.pallas_call(
        functools.partial(_gram_body, B=B, T=T),
        grid_spec=pltpu.PrefetchScalarGridSpec(
            num_scalar_prefetch=3,
            grid=(T,),
            in_specs=[
                pl.BlockSpec((B, d), row_map),
                pl.BlockSpec((B, d), col_map),
                pl.BlockSpec((1, B, 1), row3_map),
                pl.BlockSpec((1, B, 1), row3_map),
            ],
            out_specs=[
                pl.BlockSpec((1, B, B), tile_map),
                pl.BlockSpec((NB, B, 1), whole3_map),
            ],
            scratch_shapes=[pltpu.VMEM((NB, B, 1), f32)],
        ),
        out_shape=[
            jax.ShapeDtypeStruct((T, B, B), f32),
            jax.ShapeDtypeStruct((NB, B, 1), f32),
        ],
    )(ti, tj, num_arr, v_n, v_n, rs, re)

    matvec = pl.pallas_call(
        functools.partial(_matvec_body, T=T),
        grid_spec=pltpu.PrefetchScalarGridSpec(
            num_scalar_prefetch=3,
            grid=(T,),
            in_specs=[
                pl.BlockSpec((1, B, B), tile_map),
                pl.BlockSpec((NB, B, 1), whole3_map),
                pl.BlockSpec((NB, B, 1), whole3_map),
            ],
            out_specs=pl.BlockSpec((NB, B, 1), whole3_map),
            scratch_shapes=[pltpu.VMEM((NB, B, 1), f32)],
        ),
        out_shape=jax.ShapeDtypeStruct((NB, B, 1), f32),
    )

    ones = jnp.ones((NB, B, 1), f32)
    b_s = matvec(ti, tj, num_arr, k0, a1, ones)
    a_s = matvec(ti, tj, num_arr, k0, b_s, a1)
    for _ in range(3):
        b_s = matvec(ti, tj, num_arr, k0, a_s, b_s)
        a_s = matvec(ti, tj, num_arr, k0, b_s, a_s)
    b_s = matvec(ti, tj, num_arr, k0, a_s, b_s)

    out = pl.pallas_call(
        functools.partial(_out_body, B=B, T=T),
        grid_spec=pltpu.PrefetchScalarGridSpec(
            num_scalar_prefetch=3,
            grid=(T,),
            in_specs=[
                pl.BlockSpec((1, B, B), tile_map),
                pl.BlockSpec((NB, B, 1), whole3_map),
                pl.BlockSpec((B, d), col_map),
            ],
            out_specs=pl.BlockSpec((B, d), row_map),
            scratch_shapes=[pltpu.VMEM((B, 1), f32)],
        ),
        out_shape=jax.ShapeDtypeStruct((N, d), f32),
    )(ti, tj, num_arr, k0, b_s, v_feats)

    return out
